# baseline (speedup 1.0000x reference)
import functools

import jax
import jax.numpy as jnp
from jax import lax
from jax.experimental import pallas as pl
from jax.experimental.pallas import tpu as pltpu
from jax.experimental.pallas import tpu_sc as plsc


def kernel(x, emb_weight):
    seq_len = x.shape[1]
    emb_dim = emb_weight.shape[1]

    info = plsc.get_sparse_core_info()
    num_cores, num_subcores = info.num_cores, info.num_subcores
    num_workers = num_cores * num_subcores
    rows_per_worker = seq_len // num_workers
    assert rows_per_worker * num_workers == seq_len

    mesh = plsc.VectorSubcoreMesh(core_axis_name="c", subcore_axis_name="s")

    chunk_rows = 16
    num_chunks = rows_per_worker // chunk_rows
    assert chunk_rows * num_chunks == rows_per_worker
    nbuf = 7

    @functools.partial(
        pl.kernel,
        mesh=mesh,
        out_type=jax.ShapeDtypeStruct((seq_len, emb_dim), jnp.float32),
        scratch_types=(
            [pltpu.VMEM((chunk_rows, emb_dim), jnp.float32)] * nbuf
            + [pltpu.SemaphoreType.DMA] * (2 * nbuf)
        ),
    )
    def copy_rows(table_hbm, out_hbm, *scratch):
        bufs = scratch[:nbuf]
        sem_in = scratch[nbuf:2 * nbuf]
        sem_out = scratch[2 * nbuf:]
        wid = lax.axis_index("s") * num_cores + lax.axis_index("c")
        base = wid * chunk_rows
        stride = num_workers * chunk_rows

        def load(i):
            return pltpu.async_copy(
                table_hbm.at[pl.ds(base + i * stride, chunk_rows)],
                bufs[i % nbuf], sem_in[i % nbuf])

        def store(i):
            return pltpu.async_copy(
                bufs[i % nbuf],
                out_hbm.at[pl.ds(base + i * stride, chunk_rows)],
                sem_out[i % nbuf])

        loads = [None] * num_chunks
        stores = [None] * num_chunks
        for k in range(min(nbuf - 1, num_chunks)):
            loads[k] = load(k)
        for i in range(num_chunks):
            k = i + nbuf - 1
            if k < num_chunks:
                if i >= 1:
                    stores[i - 1].wait()
                loads[k] = load(k)
            loads[i].wait()
            stores[i] = store(i)
        for i in range(max(0, num_chunks - nbuf), num_chunks):
            stores[i].wait()

    return copy_rows(emb_weight)[None]


# speedup vs baseline: 1.0078x; 1.0078x over previous
import functools

import jax
import jax.numpy as jnp
from jax import lax
from jax.experimental import pallas as pl
from jax.experimental.pallas import tpu as pltpu
from jax.experimental.pallas import tpu_sc as plsc


def kernel(x, emb_weight):
    seq_len = x.shape[1]
    emb_dim = emb_weight.shape[1]

    info = plsc.get_sparse_core_info()
    num_cores, num_subcores = info.num_cores, info.num_subcores
    num_workers = num_cores * num_subcores
    rows_per_worker = seq_len // num_workers
    assert rows_per_worker * num_workers == seq_len

    mesh = plsc.VectorSubcoreMesh(core_axis_name="c", subcore_axis_name="s")

    chunk_rows = 32
    num_chunks = rows_per_worker // chunk_rows
    assert chunk_rows * num_chunks == rows_per_worker
    nbuf = 3

    @functools.partial(
        pl.kernel,
        mesh=mesh,
        out_type=jax.ShapeDtypeStruct((seq_len, emb_dim), jnp.float32),
        scratch_types=(
            [pltpu.VMEM((chunk_rows, emb_dim), jnp.float32)] * nbuf
            + [pltpu.SemaphoreType.DMA] * (2 * nbuf)
        ),
    )
    def copy_rows(table_hbm, out_hbm, *scratch):
        bufs = scratch[:nbuf]
        sem_in = scratch[nbuf:2 * nbuf]
        sem_out = scratch[2 * nbuf:]
        wid = lax.axis_index("s") * num_cores + lax.axis_index("c")
        base = wid * chunk_rows
        stride = num_workers * chunk_rows

        def load(i):
            return pltpu.async_copy(
                table_hbm.at[pl.ds(base + i * stride, chunk_rows)],
                bufs[i % nbuf], sem_in[i % nbuf])

        def store(i):
            return pltpu.async_copy(
                bufs[i % nbuf],
                out_hbm.at[pl.ds(base + i * stride, chunk_rows)],
                sem_out[i % nbuf])

        loads = [None] * num_chunks
        stores = [None] * num_chunks
        for k in range(min(nbuf - 1, num_chunks)):
            loads[k] = load(k)
        for i in range(num_chunks):
            k = i + nbuf - 1
            if k < num_chunks:
                if i >= 1:
                    stores[i - 1].wait()
                loads[k] = load(k)
            loads[i].wait()
            stores[i] = store(i)
        for i in range(max(0, num_chunks - nbuf), num_chunks):
            stores[i].wait()

    return copy_rows(emb_weight)[None]
